# Initial kernel scaffold; baseline (speedup 1.0000x reference)
import jax
import jax.numpy as jnp
from jax.experimental import pallas as pl


def kernel(sensor_indices, embedding_table):
    raise NotImplementedError("write your pallas kernel here")


# speedup vs baseline: 1.2647x; 1.2647x over previous
import functools

import jax
import jax.numpy as jnp
from jax import lax
from jax.experimental import pallas as pl
from jax.experimental.pallas import tpu as pltpu
from jax.experimental.pallas import tpu_sc as plsc

NUM_ROWS = 21
D_MODEL = 128

_NC = 2
_NS = 16
_NW = _NC * _NS

_B = 4096 * 100
_B_PER_W = _B // _NW
_CHUNK = 128
_N_CHUNKS = _B_PER_W // _CHUNK


def _emb_body(idx_hbm, table_hbm, out_hbm, idx_v, rows_v, sem):
    wid = lax.axis_index("s") * _NC + lax.axis_index("c")

    pltpu.sync_copy(idx_hbm.at[wid], idx_v)

    def chunk(j, carry):
        pltpu.async_copy(table_hbm.at[idx_v.at[j]], rows_v, sem).wait()
        pltpu.sync_copy(rows_v, out_hbm.at[wid, j])
        return carry

    lax.fori_loop(0, _N_CHUNKS, chunk, 0)


_emb = functools.partial(
    pl.kernel,
    out_type=jax.ShapeDtypeStruct((_NW, _N_CHUNKS, _CHUNK, D_MODEL), jnp.float32),
    mesh=plsc.VectorSubcoreMesh(core_axis_name="c", subcore_axis_name="s"),
    scratch_types=[
        pltpu.VMEM((_N_CHUNKS, _CHUNK), jnp.int32),
        pltpu.VMEM((_CHUNK, D_MODEL), jnp.float32),
        pltpu.SemaphoreType.DMA,
    ],
)(_emb_body)


def kernel(sensor_indices, embedding_table):
    idx = sensor_indices.reshape(_NW, _N_CHUNKS, _CHUNK).astype(jnp.int32)
    out = _emb(idx, embedding_table)
    return out.reshape(sensor_indices.shape + (D_MODEL,))


# speedup vs baseline: 1.2662x; 1.0012x over previous
import functools

import jax
import jax.numpy as jnp
from jax import lax
from jax.experimental import pallas as pl
from jax.experimental.pallas import tpu as pltpu
from jax.experimental.pallas import tpu_sc as plsc

NUM_ROWS = 21
D_MODEL = 128

_NC = 2
_NS = 16
_NW = _NC * _NS

_B = 4096 * 100
_B_PER_W = _B // _NW
_CHUNK = 128
_N_CHUNKS = _B_PER_W // _CHUNK
_N_PAIRS = _N_CHUNKS // 2


def _emb_body(idx_hbm, table_hbm, out_hbm, idx_v, buf0, buf1, g0, g1, w0, w1):
    wid = lax.axis_index("s") * _NC + lax.axis_index("c")

    pltpu.sync_copy(idx_hbm.at[wid], idx_v)

    def fire_gather(s, buf, sem):
        pltpu.make_async_copy(table_hbm.at[idx_v.at[s]], buf, sem).start()

    def wait_gather(buf, sem):
        pltpu.make_async_copy(table_hbm.at[idx_v.at[0]], buf, sem).wait()

    def fire_write(s, buf, sem):
        pltpu.make_async_copy(buf, out_hbm.at[wid, s], sem).start()

    def wait_write(buf, sem):
        pltpu.make_async_copy(buf, out_hbm.at[wid, 0], sem).wait()

    def cycle(i, first, last_static=False):
        s0 = 2 * i
        s1 = s0 + 1
        wait_gather(buf0, g0)
        fire_write(s0, buf0, w0)
        if not first:
            wait_write(buf1, w1)
        fire_gather(s1, buf1, g1)
        wait_gather(buf1, g1)
        fire_write(s1, buf1, w1)
        wait_write(buf0, w0)

        def regather():
            fire_gather(s0 + 2, buf0, g0)

        if not last_static:
            pl.when(i < _N_PAIRS - 1)(regather)

    fire_gather(0, buf0, g0)
    cycle(0, first=True)

    def body(i, carry):
        cycle(i, first=False)
        return carry

    lax.fori_loop(1, _N_PAIRS, body, 0)
    wait_write(buf1, w1)


_emb = functools.partial(
    pl.kernel,
    out_type=jax.ShapeDtypeStruct((_NW, _N_CHUNKS, _CHUNK, D_MODEL), jnp.float32),
    mesh=plsc.VectorSubcoreMesh(core_axis_name="c", subcore_axis_name="s"),
    scratch_types=[
        pltpu.VMEM((_N_CHUNKS, _CHUNK), jnp.int32),
        pltpu.VMEM((_CHUNK, D_MODEL), jnp.float32),
        pltpu.VMEM((_CHUNK, D_MODEL), jnp.float32),
        pltpu.SemaphoreType.DMA,
        pltpu.SemaphoreType.DMA,
        pltpu.SemaphoreType.DMA,
        pltpu.SemaphoreType.DMA,
    ],
)(_emb_body)


def kernel(sensor_indices, embedding_table):
    idx = sensor_indices.reshape(_NW, _N_CHUNKS, _CHUNK).astype(jnp.int32)
    out = _emb(idx, embedding_table)
    return out.reshape(sensor_indices.shape + (D_MODEL,))


# speedup vs baseline: 4.0544x; 3.2019x over previous
import functools

import jax
import jax.numpy as jnp
from jax import lax
from jax.experimental import pallas as pl
from jax.experimental.pallas import tpu as pltpu
from jax.experimental.pallas import tpu_sc as plsc

NUM_ROWS = 21
D_MODEL = 128

_NC = 2
_NS = 16
_NW = _NC * _NS

_B = 4096 * 100
_B_PER_W = _B // _NW
_CHUNK = 128
_N_CHUNKS = _B_PER_W // _CHUNK
_N_PAIRS = _N_CHUNKS // 2


def _emb_body(idx_hbm, table_hbm, out_hbm, idx_v, table_sp, buf0, buf1,
              g0, g1, w0, w1):
    sid = lax.axis_index("s")
    wid = sid * _NC + lax.axis_index("c")

    pl.when(sid == 0)(lambda: pltpu.sync_copy(table_hbm, table_sp))

    pltpu.sync_copy(idx_hbm.at[wid], idx_v)
    plsc.subcore_barrier()

    def fire_gather(s, buf, sem):
        pltpu.make_async_copy(table_sp.at[idx_v.at[s]], buf, sem).start()

    def wait_gather(buf, sem):
        pltpu.make_async_copy(table_sp.at[idx_v.at[0]], buf, sem).wait()

    def fire_write(s, buf, sem):
        pltpu.make_async_copy(buf, out_hbm.at[wid, s], sem).start()

    def wait_write(buf, sem):
        pltpu.make_async_copy(buf, out_hbm.at[wid, 0], sem).wait()

    def cycle(i, first, last_static=False):
        s0 = 2 * i
        s1 = s0 + 1
        wait_gather(buf0, g0)
        fire_write(s0, buf0, w0)
        if not first:
            wait_write(buf1, w1)
        fire_gather(s1, buf1, g1)
        wait_gather(buf1, g1)
        fire_write(s1, buf1, w1)
        wait_write(buf0, w0)

        def regather():
            fire_gather(s0 + 2, buf0, g0)

        if not last_static:
            pl.when(i < _N_PAIRS - 1)(regather)

    fire_gather(0, buf0, g0)
    cycle(0, first=True)

    def body(i, carry):
        cycle(i, first=False)
        return carry

    lax.fori_loop(1, _N_PAIRS, body, 0)
    wait_write(buf1, w1)


_emb = functools.partial(
    pl.kernel,
    out_type=jax.ShapeDtypeStruct((_NW, _N_CHUNKS, _CHUNK, D_MODEL), jnp.float32),
    mesh=plsc.VectorSubcoreMesh(core_axis_name="c", subcore_axis_name="s"),
    scratch_types=[
        pltpu.VMEM((_N_CHUNKS, _CHUNK), jnp.int32),
        pltpu.VMEM_SHARED((NUM_ROWS, D_MODEL), jnp.float32),
        pltpu.VMEM((_CHUNK, D_MODEL), jnp.float32),
        pltpu.VMEM((_CHUNK, D_MODEL), jnp.float32),
        pltpu.SemaphoreType.DMA,
        pltpu.SemaphoreType.DMA,
        pltpu.SemaphoreType.DMA,
        pltpu.SemaphoreType.DMA,
    ],
)(_emb_body)


def kernel(sensor_indices, embedding_table):
    idx = sensor_indices.reshape(_NW, _N_CHUNKS, _CHUNK).astype(jnp.int32)
    out = _emb(idx, embedding_table)
    return out.reshape(sensor_indices.shape + (D_MODEL,))


# speedup vs baseline: 17.0748x; 4.2114x over previous
import functools

import jax
import jax.numpy as jnp
from jax import lax
from jax.experimental import pallas as pl
from jax.experimental.pallas import tpu as pltpu
from jax.experimental.pallas import tpu_sc as plsc

NUM_ROWS = 21
D_MODEL = 128

_NC = 2
_NS = 16
_NW = _NC * _NS

_B = 4096 * 100
_B_PER_W = _B // _NW
_CHUNK = 128
_N_CHUNKS = _B_PER_W // _CHUNK
_N_PAIRS = _N_CHUNKS // 2


def _emb_body(idx_hbm, table_hbm, out_hbm, idx_v, table_sp, buf0, buf1,
              g0, g1, w0, w1):
    sid = lax.axis_index("s")
    wid = sid * _NC + lax.axis_index("c")

    pl.when(sid == 0)(lambda: pltpu.sync_copy(table_hbm, table_sp))

    pltpu.sync_copy(idx_hbm.at[wid], idx_v)
    plsc.subcore_barrier()

    def fire_gather(s, buf, sem):
        pltpu.make_async_copy(table_sp.at[idx_v.at[s]], buf, sem).start()

    def wait_gather(buf, sem):
        pltpu.make_async_copy(table_sp.at[idx_v.at[0]], buf, sem).wait()

    def fire_write(s, buf, sem):
        pltpu.make_async_copy(buf, out_hbm.at[wid, s], sem).start()

    def wait_write(buf, sem):
        pltpu.make_async_copy(buf, out_hbm.at[wid, 0], sem).wait()

    def cycle(i, first, last_static=False):
        s0 = 2 * i
        s1 = s0 + 1
        wait_gather(buf0, g0)
        fire_write(s0, buf0, w0)
        if not first:
            wait_write(buf1, w1)
        fire_gather(s1, buf1, g1)
        wait_gather(buf1, g1)
        fire_write(s1, buf1, w1)
        wait_write(buf0, w0)

        def regather():
            fire_gather(s0 + 2, buf0, g0)

        if not last_static:
            pl.when(i < _N_PAIRS - 1)(regather)

    fire_gather(0, buf0, g0)
    cycle(0, first=True)

    def body(i, carry):
        cycle(i, first=False)
        return carry

    lax.fori_loop(1, _N_PAIRS, body, 0)
    wait_write(buf1, w1)


_emb = functools.partial(
    pl.kernel,
    out_type=jax.ShapeDtypeStruct((_NW, _N_CHUNKS, _CHUNK, D_MODEL), jnp.float32),
    mesh=plsc.VectorSubcoreMesh(core_axis_name="c", subcore_axis_name="s"),
    scratch_types=[
        pltpu.VMEM((_N_CHUNKS, _CHUNK), jnp.int32),
        pltpu.VMEM_SHARED((NUM_ROWS, D_MODEL), jnp.float32),
        pltpu.VMEM((_CHUNK, D_MODEL), jnp.float32),
        pltpu.VMEM((_CHUNK, D_MODEL), jnp.float32),
        pltpu.SemaphoreType.DMA,
        pltpu.SemaphoreType.DMA,
        pltpu.SemaphoreType.DMA,
        pltpu.SemaphoreType.DMA,
    ],
)(_emb_body)


def kernel(sensor_indices, embedding_table):
    b, t = sensor_indices.shape
    idx = sensor_indices.T.reshape(_NW, _N_CHUNKS, _CHUNK).astype(jnp.int32)
    out = _emb(idx, embedding_table)
    return out.reshape(t, b, D_MODEL).transpose(1, 0, 2)


# speedup vs baseline: 18.1293x; 1.0618x over previous
import functools

import jax
import jax.numpy as jnp
from jax import lax
from jax.experimental import pallas as pl
from jax.experimental.pallas import tpu as pltpu
from jax.experimental.pallas import tpu_sc as plsc

NUM_ROWS = 21
D_MODEL = 128

_NC = 2
_NS = 16
_NW = _NC * _NS

_B = 4096 * 100
_B_PER_W = _B // _NW
_CHUNK = 128
_N_CHUNKS = _B_PER_W // _CHUNK
_NBUF = 4
_N_OUTER = _N_CHUNKS // _NBUF


def _emb_body(idx_hbm, table_hbm, out_hbm, idx_v, table_sp,
              b0, b1, b2, b3, gs0, gs1, gs2, gs3, ws0, ws1, ws2, ws3):
    sid = lax.axis_index("s")
    wid = sid * _NC + lax.axis_index("c")
    bufs = (b0, b1, b2, b3)
    gs = (gs0, gs1, gs2, gs3)
    ws = (ws0, ws1, ws2, ws3)

    pl.when(sid == 0)(lambda: pltpu.sync_copy(table_hbm, table_sp))

    pltpu.sync_copy(idx_hbm.at[wid], idx_v)
    plsc.subcore_barrier()

    def fire_gather(s, buf, sem):
        pltpu.make_async_copy(table_sp.at[idx_v.at[s]], buf, sem).start()

    def wait_gather(buf, sem):
        pltpu.make_async_copy(table_sp.at[idx_v.at[0]], buf, sem).wait()

    def fire_write(s, buf, sem):
        pltpu.make_async_copy(buf, out_hbm.at[wid, s], sem).start()

    def wait_write(buf, sem):
        pltpu.make_async_copy(buf, out_hbm.at[wid, 0], sem).wait()

    def body(i, carry):
        for b in range(_NBUF):
            j = _NBUF * i + b
            kb = (b + 2) % _NBUF
            wait_gather(bufs[b], gs[b])
            fire_write(j, bufs[b], ws[b])
            if b < 2:
                pl.when(i >= 1)(lambda: wait_write(bufs[kb], ws[kb]))
                fire_gather(j + 2, bufs[kb], gs[kb])
            else:
                def wait_then_fire(kb=kb, j=j):
                    wait_write(bufs[kb], ws[kb])
                    fire_gather(j + 2, bufs[kb], gs[kb])
                pl.when(i < _N_OUTER - 1)(wait_then_fire)
        return carry

    fire_gather(0, bufs[0], gs[0])
    fire_gather(1, bufs[1], gs[1])
    lax.fori_loop(0, _N_OUTER, body, 0)
    for b in range(_NBUF):
        wait_write(bufs[b], ws[b])


_emb = functools.partial(
    pl.kernel,
    out_type=jax.ShapeDtypeStruct((_NW, _N_CHUNKS, _CHUNK, D_MODEL), jnp.float32),
    mesh=plsc.VectorSubcoreMesh(core_axis_name="c", subcore_axis_name="s"),
    scratch_types=[
        pltpu.VMEM((_N_CHUNKS, _CHUNK), jnp.int32),
        pltpu.VMEM_SHARED((NUM_ROWS, D_MODEL), jnp.float32),
        pltpu.VMEM((_CHUNK, D_MODEL), jnp.float32),
        pltpu.VMEM((_CHUNK, D_MODEL), jnp.float32),
        pltpu.VMEM((_CHUNK, D_MODEL), jnp.float32),
        pltpu.VMEM((_CHUNK, D_MODEL), jnp.float32),
        pltpu.SemaphoreType.DMA,
        pltpu.SemaphoreType.DMA,
        pltpu.SemaphoreType.DMA,
        pltpu.SemaphoreType.DMA,
        pltpu.SemaphoreType.DMA,
        pltpu.SemaphoreType.DMA,
        pltpu.SemaphoreType.DMA,
        pltpu.SemaphoreType.DMA,
    ],
)(_emb_body)


def kernel(sensor_indices, embedding_table):
    b, t = sensor_indices.shape
    idx = sensor_indices.T.reshape(_NW, _N_CHUNKS, _CHUNK).astype(jnp.int32)
    out = _emb(idx, embedding_table)
    return out.reshape(t, b, D_MODEL).transpose(1, 0, 2)


# speedup vs baseline: 18.3930x; 1.0145x over previous
import functools

import jax
import jax.numpy as jnp
from jax import lax
from jax.experimental import pallas as pl
from jax.experimental.pallas import tpu as pltpu
from jax.experimental.pallas import tpu_sc as plsc

NUM_ROWS = 21
D_MODEL = 128

_NC = 2
_NS = 16
_NW = _NC * _NS

_B = 4096 * 100
_B_PER_W = _B // _NW
_CHUNK = 128
_N_CHUNKS = _B_PER_W // _CHUNK
_NBUF = 5
_FIRE_AHEAD = 3
_N_OUTER = _N_CHUNKS // _NBUF


def _emb_body(idx_hbm, table_hbm, out_hbm, idx_v, table_sp,
              b0, b1, b2, b3, b4,
              gs0, gs1, gs2, gs3, gs4, ws0, ws1, ws2, ws3, ws4):
    sid = lax.axis_index("s")
    wid = sid * _NC + lax.axis_index("c")
    bufs = (b0, b1, b2, b3, b4)
    gs = (gs0, gs1, gs2, gs3, gs4)
    ws = (ws0, ws1, ws2, ws3, ws4)

    pl.when(sid == 0)(lambda: pltpu.sync_copy(table_hbm, table_sp))

    pltpu.sync_copy(idx_hbm.at[wid], idx_v)
    plsc.subcore_barrier()

    def fire_gather(s, buf, sem):
        pltpu.make_async_copy(table_sp.at[idx_v.at[s]], buf, sem).start()

    def wait_gather(buf, sem):
        pltpu.make_async_copy(table_sp.at[idx_v.at[0]], buf, sem).wait()

    def fire_write(s, buf, sem):
        pltpu.make_async_copy(buf, out_hbm.at[wid, s], sem).start()

    def wait_write(buf, sem):
        pltpu.make_async_copy(buf, out_hbm.at[wid, 0], sem).wait()

    def body(i, carry):
        for b in range(_NBUF):
            j = _NBUF * i + b
            kb = (b + _FIRE_AHEAD) % _NBUF
            k = j + _FIRE_AHEAD
            wait_gather(bufs[b], gs[b])
            fire_write(j, bufs[b], ws[b])
            if b < _NBUF - _FIRE_AHEAD:
                pl.when(i >= 1)(lambda: wait_write(bufs[kb], ws[kb]))
                fire_gather(k, bufs[kb], gs[kb])
            else:
                def wait_then_fire(kb=kb, k=k):
                    wait_write(bufs[kb], ws[kb])
                    fire_gather(k, bufs[kb], gs[kb])
                pl.when(i < _N_OUTER - 1)(wait_then_fire)
        return carry

    for s in range(_FIRE_AHEAD):
        fire_gather(s, bufs[s], gs[s])
    lax.fori_loop(0, _N_OUTER, body, 0)
    for b in range(_NBUF):
        wait_write(bufs[b], ws[b])


_emb = functools.partial(
    pl.kernel,
    out_type=jax.ShapeDtypeStruct((_NW, _N_CHUNKS, _CHUNK, D_MODEL), jnp.float32),
    mesh=plsc.VectorSubcoreMesh(core_axis_name="c", subcore_axis_name="s"),
    scratch_types=[
        pltpu.VMEM((_N_CHUNKS, _CHUNK), jnp.int32),
        pltpu.VMEM_SHARED((NUM_ROWS, D_MODEL), jnp.float32),
        pltpu.VMEM((_CHUNK, D_MODEL), jnp.float32),
        pltpu.VMEM((_CHUNK, D_MODEL), jnp.float32),
        pltpu.VMEM((_CHUNK, D_MODEL), jnp.float32),
        pltpu.VMEM((_CHUNK, D_MODEL), jnp.float32),
        pltpu.VMEM((_CHUNK, D_MODEL), jnp.float32),
        pltpu.SemaphoreType.DMA,
        pltpu.SemaphoreType.DMA,
        pltpu.SemaphoreType.DMA,
        pltpu.SemaphoreType.DMA,
        pltpu.SemaphoreType.DMA,
        pltpu.SemaphoreType.DMA,
        pltpu.SemaphoreType.DMA,
        pltpu.SemaphoreType.DMA,
        pltpu.SemaphoreType.DMA,
        pltpu.SemaphoreType.DMA,
    ],
)(_emb_body)


def kernel(sensor_indices, embedding_table):
    b, t = sensor_indices.shape
    idx = sensor_indices.T.reshape(_NW, _N_CHUNKS, _CHUNK).astype(jnp.int32)
    out = _emb(idx, embedding_table)
    return out.reshape(t, b, D_MODEL).transpose(1, 0, 2)
